# Initial kernel scaffold; baseline (speedup 1.0000x reference)
import jax
import jax.numpy as jnp
from jax.experimental import pallas as pl


def kernel(pred, target):
    raise NotImplementedError("write your pallas kernel here")


# speedup vs baseline: 3.8104x; 3.8104x over previous
import jax
import jax.numpy as jnp
from jax.experimental import pallas as pl
from jax.experimental.pallas import tpu as pltpu

N = 2048
BLK = 256
NBLK = N // BLK


def _rank_sort_body(vrows_ref, vcols_ref, pos_ref, s_ref):
    for r in range(2):
        vrow = vrows_ref[r : r + 1, :]

        def count_blk(b, _):
            vi = vcols_ref[pl.ds(b * BLK, BLK), r : r + 1]
            jj = jax.lax.broadcasted_iota(jnp.int32, (BLK, N), 1)
            ii = jax.lax.broadcasted_iota(jnp.int32, (BLK, N), 0) + b * BLK
            before = (vrow > vi) | ((vrow == vi) & (jj < ii))
            cnt = jnp.sum(jnp.where(before, 1.0, 0.0), axis=1, keepdims=True)
            pos_ref[pl.ds(b * BLK, BLK), r : r + 1] = cnt
            return 0

        jax.lax.fori_loop(0, NBLK, count_blk, 0)

        def gather_blk(b, acc):
            posb = pos_ref[pl.ds(b * BLK, BLK), r : r + 1].astype(jnp.int32)
            vi = vcols_ref[pl.ds(b * BLK, BLK), r : r + 1]
            kk = jax.lax.broadcasted_iota(jnp.int32, (BLK, N), 1)
            e = jnp.where(posb == kk, vi, 0.0)
            return acc + jnp.sum(e, axis=0, keepdims=True)

        s_ref[r : r + 1, :] = jax.lax.fori_loop(
            0, NBLK, gather_blk, jnp.zeros((1, N), jnp.float32)
        )


def _pav_body(s_ref, sol_ref, means, counts):
    n = N
    fn = jnp.float32(n)

    def trip(_, carry):
        k, sp, top_m, top_c = carry
        prev = jnp.maximum(sp - 1, 0)
        pm = means[prev]
        pc = counts[prev]
        can_merge = (sp > 0) & (top_m > pm)
        can_push = k < n
        mm = (top_m * top_c + pm * pc) / (top_c + pc)
        mc = top_c + pc
        zk = s_ref[0, 0, jnp.minimum(k, n - 1)] - fn + k.astype(jnp.float32)
        st_idx = jnp.where(can_merge, prev, jnp.where(can_push, sp, n))
        means[st_idx] = jnp.where(can_merge, mm, top_m)
        counts[st_idx] = jnp.where(can_merge, mc, top_c)
        new_top_m = jnp.where(can_merge, mm, jnp.where(can_push, zk, top_m))
        new_top_c = jnp.where(can_merge, mc, jnp.where(can_push, 1.0, top_c))
        new_sp = sp + jnp.where(can_merge, -1, jnp.where(can_push, 1, 0))
        new_k = k + jnp.where(~can_merge & can_push, 1, 0)
        return new_k, new_sp, new_top_m, new_top_c

    z0 = s_ref[0, 0, 0] - fn
    k, sp, top_m, top_c = jax.lax.fori_loop(
        0, 2 * n, trip, (jnp.int32(1), jnp.int32(0), z0, jnp.float32(1.0))
    )
    means[sp] = top_m
    counts[sp] = top_c

    def expand(kk, carry):
        b, rem = carry
        sol_ref[0, 0, kk] = s_ref[0, 0, kk] - means[b]
        rem = rem - 1.0
        nb = jnp.where(rem <= 0.0, b + 1, b)
        nrem = jnp.where(rem <= 0.0, counts[jnp.minimum(nb, n)], rem)
        return nb, nrem

    jax.lax.fori_loop(0, n, expand, (jnp.int32(0), counts[0]))


def _loss_body(sol_ref, pos_ref, out_ref):
    solp = sol_ref[0:1, :]
    solt = sol_ref[1:2, :]
    fn = jnp.float32(N)
    mp = jnp.sum(solp) / fn
    mt = jnp.sum(solt) / fn
    vp = jnp.sum((solp - mp) ** 2)
    vt = jnp.sum((solt - mt) ** 2)
    cp = solp - mp
    ct = solt - mt

    def dot_blk(b, acc):
        kk = jax.lax.broadcasted_iota(jnp.int32, (BLK, N), 1)
        pp = pos_ref[pl.ds(b * BLK, BLK), 0:1].astype(jnp.int32)
        pt = pos_ref[pl.ds(b * BLK, BLK), 1:2].astype(jnp.int32)
        rp = jnp.sum(jnp.where(pp == kk, cp, 0.0), axis=1, keepdims=True)
        rt = jnp.sum(jnp.where(pt == kk, ct, 0.0), axis=1, keepdims=True)
        return acc + jnp.sum(rp * rt)

    dot = jax.lax.fori_loop(0, NBLK, dot_blk, jnp.float32(0.0))
    out_ref[0, 0] = -dot / (jnp.sqrt(vp) * jnp.sqrt(vt))


def kernel(pred, target):
    vrows = jnp.concatenate([pred, target], axis=0)
    vcols = vrows.T
    pos, s = pl.pallas_call(
        _rank_sort_body,
        out_shape=(
            jax.ShapeDtypeStruct((N, 2), jnp.float32),
            jax.ShapeDtypeStruct((2, N), jnp.float32),
        ),
    )(vrows, vcols)
    sol = pl.pallas_call(
        _pav_body,
        grid=(2,),
        in_specs=[
            pl.BlockSpec((1, 1, N), lambda i: (i, 0, 0), memory_space=pltpu.SMEM)
        ],
        out_specs=pl.BlockSpec((1, 1, N), lambda i: (i, 0, 0), memory_space=pltpu.SMEM),
        out_shape=jax.ShapeDtypeStruct((2, 1, N), jnp.float32),
        scratch_shapes=[
            pltpu.SMEM((N + 1,), jnp.float32),
            pltpu.SMEM((N + 1,), jnp.float32),
        ],
    )(s.reshape(2, 1, N))
    sol = sol.reshape(2, N)
    loss = pl.pallas_call(
        _loss_body,
        out_specs=pl.BlockSpec(memory_space=pltpu.SMEM),
        out_shape=jax.ShapeDtypeStruct((1, 1), jnp.float32),
    )(sol, pos)
    return loss[0, 0]


# speedup vs baseline: 8.1747x; 2.1454x over previous
import jax
import jax.numpy as jnp
from jax.experimental import pallas as pl
from jax.experimental.pallas import tpu as pltpu

N = 2048
BLK = 256
NBLK = N // BLK


def _rank_sort_body(vrows_ref, vcols_ref, pos_ref, s_ref):
    for r in range(2):
        vrow = vrows_ref[r : r + 1, :]

        def count_blk(b, _):
            vi = vcols_ref[pl.ds(b * BLK, BLK), r : r + 1]
            jj = jax.lax.broadcasted_iota(jnp.int32, (BLK, N), 1)
            ii = jax.lax.broadcasted_iota(jnp.int32, (BLK, N), 0) + b * BLK
            before = (vrow > vi) | ((vrow == vi) & (jj < ii))
            cnt = jnp.sum(jnp.where(before, 1.0, 0.0), axis=1, keepdims=True)
            pos_ref[pl.ds(b * BLK, BLK), r : r + 1] = cnt
            return 0

        jax.lax.fori_loop(0, NBLK, count_blk, 0)

        def gather_blk(b, acc):
            posb = pos_ref[pl.ds(b * BLK, BLK), r : r + 1].astype(jnp.int32)
            vi = vcols_ref[pl.ds(b * BLK, BLK), r : r + 1]
            kk = jax.lax.broadcasted_iota(jnp.int32, (BLK, N), 1)
            e = jnp.where(posb == kk, vi, 0.0)
            return acc + jnp.sum(e, axis=0, keepdims=True)

        s_ref[r : r + 1, :] = jax.lax.fori_loop(
            0, NBLK, gather_blk, jnp.zeros((1, N), jnp.float32)
        )


def _pav_body(s_ref, sol_ref, means, counts):
    n = N
    fn = jnp.float32(n)

    def trip(_, carry):
        k, sp, top_s, top_c = carry
        prev = jnp.maximum(sp - 1, 0)
        ps = means[prev]
        pc = counts[prev]
        can_merge = (sp > 0) & (top_s * pc > ps * top_c)
        can_push = k < n
        ms = top_s + ps
        mc = top_c + pc
        zk = s_ref[0, 0, jnp.minimum(k, n - 1)] - fn + k.astype(jnp.float32)
        st_idx = jnp.where(can_merge, prev, jnp.where(can_push, sp, n))
        means[st_idx] = jnp.where(can_merge, ms, top_s)
        counts[st_idx] = jnp.where(can_merge, mc, top_c)
        new_top_s = jnp.where(can_merge, ms, jnp.where(can_push, zk, top_s))
        new_top_c = jnp.where(can_merge, mc, jnp.where(can_push, 1.0, top_c))
        new_sp = sp + jnp.where(can_merge, -1, jnp.where(can_push, 1, 0))
        new_k = k + jnp.where(~can_merge & can_push, 1, 0)
        return new_k, new_sp, new_top_s, new_top_c

    z0 = s_ref[0, 0, 0] - fn
    k, sp, top_s, top_c = jax.lax.fori_loop(
        0, 2 * n, trip, (jnp.int32(1), jnp.int32(0), z0, jnp.float32(1.0))
    )
    means[sp] = top_s
    counts[sp] = top_c

    def blockmean(b, _):
        means[b] = means[b] / counts[b]
        return 0

    jax.lax.fori_loop(0, sp + 1, blockmean, 0)

    def expand(kk, carry):
        b, rem = carry
        sol_ref[0, 0, kk] = s_ref[0, 0, kk] - means[b]
        rem = rem - 1.0
        nb = jnp.where(rem <= 0.0, b + 1, b)
        nrem = jnp.where(rem <= 0.0, counts[jnp.minimum(nb, n)], rem)
        return nb, nrem

    jax.lax.fori_loop(0, n, expand, (jnp.int32(0), counts[0]))


def _loss_body(sol_ref, pos_ref, out_ref):
    solp = sol_ref[0:1, :]
    solt = sol_ref[1:2, :]
    fn = jnp.float32(N)
    mp = jnp.sum(solp) / fn
    mt = jnp.sum(solt) / fn
    vp = jnp.sum((solp - mp) ** 2)
    vt = jnp.sum((solt - mt) ** 2)
    cp = solp - mp
    ct = solt - mt

    def dot_blk(b, acc):
        kk = jax.lax.broadcasted_iota(jnp.int32, (BLK, N), 1)
        pp = pos_ref[pl.ds(b * BLK, BLK), 0:1].astype(jnp.int32)
        pt = pos_ref[pl.ds(b * BLK, BLK), 1:2].astype(jnp.int32)
        rp = jnp.sum(jnp.where(pp == kk, cp, 0.0), axis=1, keepdims=True)
        rt = jnp.sum(jnp.where(pt == kk, ct, 0.0), axis=1, keepdims=True)
        return acc + jnp.sum(rp * rt)

    dot = jax.lax.fori_loop(0, NBLK, dot_blk, jnp.float32(0.0))
    out_ref[0, 0] = -dot / (jnp.sqrt(vp) * jnp.sqrt(vt))


def kernel(pred, target):
    vrows = jnp.concatenate([pred, target], axis=0)
    vcols = vrows.T
    pos, s = pl.pallas_call(
        _rank_sort_body,
        out_shape=(
            jax.ShapeDtypeStruct((N, 2), jnp.float32),
            jax.ShapeDtypeStruct((2, N), jnp.float32),
        ),
    )(vrows, vcols)
    sol = pl.pallas_call(
        _pav_body,
        grid=(2,),
        in_specs=[
            pl.BlockSpec((1, 1, N), lambda i: (i, 0, 0), memory_space=pltpu.SMEM)
        ],
        out_specs=pl.BlockSpec((1, 1, N), lambda i: (i, 0, 0), memory_space=pltpu.SMEM),
        out_shape=jax.ShapeDtypeStruct((2, 1, N), jnp.float32),
        scratch_shapes=[
            pltpu.SMEM((N + 1,), jnp.float32),
            pltpu.SMEM((N + 1,), jnp.float32),
        ],
    )(s.reshape(2, 1, N))
    sol = sol.reshape(2, N)
    loss = pl.pallas_call(
        _loss_body,
        out_specs=pl.BlockSpec(memory_space=pltpu.SMEM),
        out_shape=jax.ShapeDtypeStruct((1, 1), jnp.float32),
    )(sol, pos)
    return loss[0, 0]


# speedup vs baseline: 8.1890x; 1.0017x over previous
import jax
import jax.numpy as jnp
from jax.experimental import pallas as pl
from jax.experimental.pallas import tpu as pltpu

N = 2048
BLK = 256
NBLK = N // BLK


def _rank_sort_body(vrows_ref, vcols_ref, pos_ref, s_ref):
    for r in range(2):
        vrow = vrows_ref[r : r + 1, :]

        def count_blk(b, _):
            vi = vcols_ref[pl.ds(b * BLK, BLK), r : r + 1]
            jj = jax.lax.broadcasted_iota(jnp.int32, (BLK, N), 1)
            ii = jax.lax.broadcasted_iota(jnp.int32, (BLK, N), 0) + b * BLK
            before = (vrow > vi) | ((vrow == vi) & (jj < ii))
            cnt = jnp.sum(jnp.where(before, 1.0, 0.0), axis=1, keepdims=True)
            pos_ref[pl.ds(b * BLK, BLK), r : r + 1] = cnt
            return 0

        jax.lax.fori_loop(0, NBLK, count_blk, 0)

        def gather_blk(b, acc):
            posb = pos_ref[pl.ds(b * BLK, BLK), r : r + 1].astype(jnp.int32)
            vi = vcols_ref[pl.ds(b * BLK, BLK), r : r + 1]
            kk = jax.lax.broadcasted_iota(jnp.int32, (BLK, N), 1)
            e = jnp.where(posb == kk, vi, 0.0)
            return acc + jnp.sum(e, axis=0, keepdims=True)

        s_ref[r : r + 1, :] = jax.lax.fori_loop(
            0, NBLK, gather_blk, jnp.zeros((1, N), jnp.float32)
        )


def _pav_body(s_ref, sol_ref, means, counts):
    n = N
    fn = jnp.float32(n)

    def step(r, k, sp, top_s, top_c):
        prev = jnp.maximum(sp - 1, 0)
        ps = means[r, prev]
        pc = counts[r, prev]
        can_merge = (sp > 0) & (top_s * pc > ps * top_c)
        can_push = k < n
        ms = top_s + ps
        mc = top_c + pc
        zk = s_ref[r, jnp.minimum(k, n - 1)] - fn + k.astype(jnp.float32)
        st_idx = jnp.where(can_merge, prev, jnp.where(can_push, sp, n))
        means[r, st_idx] = jnp.where(can_merge, ms, top_s)
        counts[r, st_idx] = jnp.where(can_merge, mc, top_c)
        new_top_s = jnp.where(can_merge, ms, jnp.where(can_push, zk, top_s))
        new_top_c = jnp.where(can_merge, mc, jnp.where(can_push, 1.0, top_c))
        new_sp = sp + jnp.where(can_merge, -1, jnp.where(can_push, 1, 0))
        new_k = k + jnp.where(~can_merge & can_push, 1, 0)
        return new_k, new_sp, new_top_s, new_top_c

    def trip(_, carry):
        a, b = carry
        return step(0, *a), step(1, *b)

    init0 = (jnp.int32(1), jnp.int32(0), s_ref[0, 0] - fn, jnp.float32(1.0))
    init1 = (jnp.int32(1), jnp.int32(0), s_ref[1, 0] - fn, jnp.float32(1.0))
    (k0, sp0, ts0, tc0), (k1, sp1, ts1, tc1) = jax.lax.fori_loop(
        0, 2 * n, trip, (init0, init1)
    )
    means[0, sp0] = ts0
    counts[0, sp0] = tc0
    means[1, sp1] = ts1
    counts[1, sp1] = tc1

    def blockmean(b, _):
        i0 = jnp.minimum(b, sp0)
        v0 = means[0, i0]
        means[0, i0] = jnp.where(b <= sp0, v0 / counts[0, i0], v0)
        i1 = jnp.minimum(b, sp1)
        v1 = means[1, i1]
        means[1, i1] = jnp.where(b <= sp1, v1 / counts[1, i1], v1)
        return 0

    jax.lax.fori_loop(0, jnp.maximum(sp0, sp1) + 1, blockmean, 0)

    def expand(kk, carry):
        b0, rem0, b1, rem1 = carry
        sol_ref[0, kk] = s_ref[0, kk] - means[0, b0]
        sol_ref[1, kk] = s_ref[1, kk] - means[1, b1]
        rem0 = rem0 - 1.0
        rem1 = rem1 - 1.0
        nb0 = jnp.where(rem0 <= 0.0, b0 + 1, b0)
        nrem0 = jnp.where(rem0 <= 0.0, counts[0, jnp.minimum(nb0, n)], rem0)
        nb1 = jnp.where(rem1 <= 0.0, b1 + 1, b1)
        nrem1 = jnp.where(rem1 <= 0.0, counts[1, jnp.minimum(nb1, n)], rem1)
        return nb0, nrem0, nb1, nrem1

    jax.lax.fori_loop(
        0, n, expand, (jnp.int32(0), counts[0, 0], jnp.int32(0), counts[1, 0])
    )


def _loss_body(sol_ref, pos_ref, out_ref):
    solp = sol_ref[0:1, :]
    solt = sol_ref[1:2, :]
    fn = jnp.float32(N)
    mp = jnp.sum(solp) / fn
    mt = jnp.sum(solt) / fn
    vp = jnp.sum((solp - mp) ** 2)
    vt = jnp.sum((solt - mt) ** 2)
    cp = solp - mp
    ct = solt - mt

    def dot_blk(b, acc):
        kk = jax.lax.broadcasted_iota(jnp.int32, (BLK, N), 1)
        pp = pos_ref[pl.ds(b * BLK, BLK), 0:1].astype(jnp.int32)
        pt = pos_ref[pl.ds(b * BLK, BLK), 1:2].astype(jnp.int32)
        rp = jnp.sum(jnp.where(pp == kk, cp, 0.0), axis=1, keepdims=True)
        rt = jnp.sum(jnp.where(pt == kk, ct, 0.0), axis=1, keepdims=True)
        return acc + jnp.sum(rp * rt)

    dot = jax.lax.fori_loop(0, NBLK, dot_blk, jnp.float32(0.0))
    out_ref[0, 0] = -dot / (jnp.sqrt(vp) * jnp.sqrt(vt))


def kernel(pred, target):
    vrows = jnp.concatenate([pred, target], axis=0)
    vcols = vrows.T
    pos, s = pl.pallas_call(
        _rank_sort_body,
        out_shape=(
            jax.ShapeDtypeStruct((N, 2), jnp.float32),
            jax.ShapeDtypeStruct((2, N), jnp.float32),
        ),
    )(vrows, vcols)
    sol = pl.pallas_call(
        _pav_body,
        in_specs=[pl.BlockSpec(memory_space=pltpu.SMEM)],
        out_specs=pl.BlockSpec(memory_space=pltpu.SMEM),
        out_shape=jax.ShapeDtypeStruct((2, N), jnp.float32),
        scratch_shapes=[
            pltpu.SMEM((2, N + 1), jnp.float32),
            pltpu.SMEM((2, N + 1), jnp.float32),
        ],
    )(s)
    loss = pl.pallas_call(
        _loss_body,
        out_specs=pl.BlockSpec(memory_space=pltpu.SMEM),
        out_shape=jax.ShapeDtypeStruct((1, 1), jnp.float32),
    )(sol, pos)
    return loss[0, 0]


# speedup vs baseline: 23.4189x; 2.8598x over previous
import jax
import jax.numpy as jnp
from jax.experimental import pallas as pl
from jax.experimental.pallas import tpu as pltpu

N = 2048
BLK = 256
NBLK = N // BLK
L = 16
BIG = 1e30


def _chunk_items(s_row, lane, lmod, lmod_f):
    lane_f = lane.astype(jnp.float32)
    y = (jnp.float32(N) - lane_f) - s_row
    cs = y
    for d in (1, 2, 4, 8):
        cs = jnp.where(lmod >= d, cs + jnp.roll(cs, d, axis=1), cs)
    cs_excl = cs - y
    fit_y = jnp.full(s_row.shape, -BIG, jnp.float32)
    for i_off in range(L):
        f = jnp.where(lmod == i_off, cs_excl, 0.0)
        for d in (1, 2, 4, 8):
            f = jnp.where(lmod >= d, f + jnp.roll(f, d, axis=1), f)
        len_f = lmod_f - jnp.float32(i_off) + 1.0
        mj = (cs - f) / len_f
        mj = jnp.where(lmod >= i_off, mj, BIG)
        for d in (1, 2, 4, 8):
            sh = jnp.where(lmod <= L - 1 - d, jnp.roll(mj, -d, axis=1), BIG)
            mj = jnp.minimum(mj, sh)
        fit_y = jnp.maximum(fit_y, jnp.where(lmod >= i_off, mj, -BIG))
    fit_z = -fit_y
    bnd = (lmod == 0) | (fit_z != jnp.roll(fit_z, 1, axis=1))
    t = jnp.where(bnd, lane, jnp.int32(1 << 20))
    sfx = t
    for d in (1, 2, 4, 8):
        sh = jnp.where(lmod <= L - 1 - d, jnp.roll(sfx, -d, axis=1), jnp.int32(1 << 20))
        sfx = jnp.minimum(sfx, sh)
    nxt_in = jnp.where(lmod <= L - 2, jnp.roll(sfx, -1, axis=1), jnp.int32(1 << 20))
    inext = jnp.minimum(nxt_in, lane - lmod + L)
    icnt = (inext - lane).astype(jnp.float32)
    isum = fit_z * icnt
    return isum, icnt, inext


def _rank_sort_body(
    vrows_ref, vcols_ref, pos_ref, s_ref, isum_ref, icnt_ref, inext_ref
):
    lane = jax.lax.broadcasted_iota(jnp.int32, (1, N), 1)
    lmod = lane & (L - 1)
    lmod_f = lmod.astype(jnp.float32)
    for r in range(2):
        vrow = vrows_ref[r : r + 1, :]

        def count_blk(b, _):
            vi = vcols_ref[pl.ds(b * BLK, BLK), r : r + 1]
            jj = jax.lax.broadcasted_iota(jnp.int32, (BLK, N), 1)
            ii = jax.lax.broadcasted_iota(jnp.int32, (BLK, N), 0) + b * BLK
            before = (vrow > vi) | ((vrow == vi) & (jj < ii))
            cnt = jnp.sum(jnp.where(before, 1.0, 0.0), axis=1, keepdims=True)
            pos_ref[pl.ds(b * BLK, BLK), r : r + 1] = cnt
            return 0

        jax.lax.fori_loop(0, NBLK, count_blk, 0)

        def gather_blk(b, acc):
            posb = pos_ref[pl.ds(b * BLK, BLK), r : r + 1].astype(jnp.int32)
            vi = vcols_ref[pl.ds(b * BLK, BLK), r : r + 1]
            kk = jax.lax.broadcasted_iota(jnp.int32, (BLK, N), 1)
            e = jnp.where(posb == kk, vi, 0.0)
            return acc + jnp.sum(e, axis=0, keepdims=True)

        s_row = jax.lax.fori_loop(
            0, NBLK, gather_blk, jnp.zeros((1, N), jnp.float32)
        )
        s_ref[r : r + 1, :] = s_row
        isum, icnt, inext = _chunk_items(s_row, lane, lmod, lmod_f)
        isum_ref[r : r + 1, :] = isum
        icnt_ref[r : r + 1, :] = icnt
        inext_ref[r : r + 1, :] = inext


def _pav_body(isum_ref, icnt_ref, inext_ref, bm_ref, bs_ref, bc_ref, nb_ref, means, counts):
    n = N

    for r in range(2):

        def cond(carry):
            return carry[4]

        def trip(carry):
            k, sp, top_s, top_c, _ = carry
            prev = jnp.maximum(sp - 1, 0)
            ps = means[r, prev]
            pc = counts[r, prev]
            can_merge = (sp > 0) & (top_s * pc > ps * top_c)
            can_push = k < n
            ms = top_s + ps
            mc = top_c + pc
            kc = jnp.minimum(k, n - 1)
            push_s = isum_ref[r, kc]
            push_c = icnt_ref[r, kc]
            push_k = inext_ref[r, kc]
            st_idx = jnp.where(can_merge, prev, jnp.where(can_push, sp, n))
            means[r, st_idx] = jnp.where(can_merge, ms, top_s)
            counts[r, st_idx] = jnp.where(can_merge, mc, top_c)
            new_top_s = jnp.where(can_merge, ms, jnp.where(can_push, push_s, top_s))
            new_top_c = jnp.where(can_merge, mc, jnp.where(can_push, push_c, top_c))
            new_sp = sp + jnp.where(can_merge, -1, jnp.where(can_push, 1, 0))
            new_k = jnp.where(can_merge, k, jnp.where(can_push, push_k, k))
            nprev = jnp.maximum(new_sp - 1, 0)
            nps = means[r, nprev]
            npc = counts[r, nprev]
            nactive = ((new_sp > 0) & (new_top_s * npc > nps * new_top_c)) | (
                new_k < n
            )
            return new_k, new_sp, new_top_s, new_top_c, nactive

        k0 = inext_ref[r, 0]
        init = (k0, jnp.int32(0), isum_ref[r, 0], icnt_ref[r, 0], k0 < n)
        k, sp, top_s, top_c, _ = jax.lax.while_loop(cond, trip, init)
        means[r, sp] = top_s
        counts[r, sp] = top_c
        nb_ref[0, r] = (sp + 1).astype(jnp.float32)

        def walk(b, start):
            c = counts[r, b]
            bm_ref[r, b] = means[r, b] / c
            bs_ref[r, b] = start
            bc_ref[r, b] = c
            return start + c

        jax.lax.fori_loop(0, sp + 1, walk, jnp.float32(0.0))


def _loss_body(s_ref, pos_ref, bm_ref, bs_ref, bc_ref, nb_ref, out_ref):
    fn = jnp.float32(N)
    nb0 = nb_ref[0, 0]
    nb1 = nb_ref[0, 1]
    kk_f = jax.lax.broadcasted_iota(jnp.int32, (BLK, N), 1).astype(jnp.float32)

    def dual_blk(b, accs):
        d0, d1 = accs
        bf = (
            jax.lax.broadcasted_iota(jnp.int32, (BLK, 1), 0) + b * BLK
        ).astype(jnp.float32)
        bs0 = bs_ref[pl.ds(b * BLK, BLK), 0:1]
        bc0 = bc_ref[pl.ds(b * BLK, BLK), 0:1]
        bm0 = bm_ref[pl.ds(b * BLK, BLK), 0:1]
        m0 = (bs0 <= kk_f) & (kk_f < bs0 + bc0) & (bf < nb0)
        d0 = d0 + jnp.sum(jnp.where(m0, bm0, 0.0), axis=0, keepdims=True)
        bs1 = bs_ref[pl.ds(b * BLK, BLK), 1:2]
        bc1 = bc_ref[pl.ds(b * BLK, BLK), 1:2]
        bm1 = bm_ref[pl.ds(b * BLK, BLK), 1:2]
        m1 = (bs1 <= kk_f) & (kk_f < bs1 + bc1) & (bf < nb1)
        d1 = d1 + jnp.sum(jnp.where(m1, bm1, 0.0), axis=0, keepdims=True)
        return d0, d1

    zero = jnp.zeros((1, N), jnp.float32)
    dual0, dual1 = jax.lax.fori_loop(0, NBLK, dual_blk, (zero, zero))
    solp = s_ref[0:1, :] - dual0
    solt = s_ref[1:2, :] - dual1
    mp = jnp.sum(solp) / fn
    mt = jnp.sum(solt) / fn
    vp = jnp.sum((solp - mp) ** 2)
    vt = jnp.sum((solt - mt) ** 2)
    cp = solp - mp
    ct = solt - mt

    def dot_blk(b, acc):
        kk = jax.lax.broadcasted_iota(jnp.int32, (BLK, N), 1)
        pp = pos_ref[pl.ds(b * BLK, BLK), 0:1].astype(jnp.int32)
        pt = pos_ref[pl.ds(b * BLK, BLK), 1:2].astype(jnp.int32)
        rp = jnp.sum(jnp.where(pp == kk, cp, 0.0), axis=1, keepdims=True)
        rt = jnp.sum(jnp.where(pt == kk, ct, 0.0), axis=1, keepdims=True)
        return acc + jnp.sum(rp * rt)

    dot = jax.lax.fori_loop(0, NBLK, dot_blk, jnp.float32(0.0))
    out_ref[0, 0] = -dot / (jnp.sqrt(vp) * jnp.sqrt(vt))


def kernel(pred, target):
    vrows = jnp.concatenate([pred, target], axis=0)
    vcols = vrows.T
    pos, s, isum, icnt, inext = pl.pallas_call(
        _rank_sort_body,
        out_shape=(
            jax.ShapeDtypeStruct((N, 2), jnp.float32),
            jax.ShapeDtypeStruct((2, N), jnp.float32),
            jax.ShapeDtypeStruct((2, N), jnp.float32),
            jax.ShapeDtypeStruct((2, N), jnp.float32),
            jax.ShapeDtypeStruct((2, N), jnp.int32),
        ),
    )(vrows, vcols)
    bm, bs, bc, nb = pl.pallas_call(
        _pav_body,
        in_specs=[pl.BlockSpec(memory_space=pltpu.SMEM)] * 3,
        out_specs=tuple([pl.BlockSpec(memory_space=pltpu.SMEM)] * 4),
        out_shape=(
            jax.ShapeDtypeStruct((2, N), jnp.float32),
            jax.ShapeDtypeStruct((2, N), jnp.float32),
            jax.ShapeDtypeStruct((2, N), jnp.float32),
            jax.ShapeDtypeStruct((1, 2), jnp.float32),
        ),
        scratch_shapes=[
            pltpu.SMEM((2, N + 1), jnp.float32),
            pltpu.SMEM((2, N + 1), jnp.float32),
        ],
    )(isum, icnt, inext)
    loss = pl.pallas_call(
        _loss_body,
        in_specs=[pl.BlockSpec(memory_space=pltpu.VMEM)] * 5
        + [pl.BlockSpec(memory_space=pltpu.SMEM)],
        out_specs=pl.BlockSpec(memory_space=pltpu.SMEM),
        out_shape=jax.ShapeDtypeStruct((1, 1), jnp.float32),
    )(s, pos, bm.T, bs.T, bc.T, nb)
    return loss[0, 0]


# speedup vs baseline: 27.3596x; 1.1683x over previous
import jax
import jax.numpy as jnp
from jax.experimental import pallas as pl
from jax.experimental.pallas import tpu as pltpu

N = 2048
BLK = 256
NBLK = N // BLK
L = 16
BIG = 1e30


def _chunk_items(s2, lane, lmod, lmod_f):
    lane_f = lane.astype(jnp.float32)
    y = (jnp.float32(N) - lane_f) - s2
    cs = y
    for d in (1, 2, 4, 8):
        cs = jnp.where(lmod >= d, cs + jnp.roll(cs, d, axis=1), cs)
    cs_excl = cs - y
    fit_y = jnp.full(s2.shape, -BIG, jnp.float32)
    for i_off in range(L):
        f = jnp.where(lmod == i_off, cs_excl, 0.0)
        for d in (1, 2, 4, 8):
            f = jnp.where(lmod >= d, f + jnp.roll(f, d, axis=1), f)
        len_f = lmod_f - jnp.float32(i_off) + 1.0
        mj = (cs - f) / len_f
        mj = jnp.where(lmod >= i_off, mj, BIG)
        for d in (1, 2, 4, 8):
            sh = jnp.where(lmod <= L - 1 - d, jnp.roll(mj, -d, axis=1), BIG)
            mj = jnp.minimum(mj, sh)
        fit_y = jnp.maximum(fit_y, jnp.where(lmod >= i_off, mj, -BIG))
    fit_z = -fit_y
    bnd = (lmod == 0) | (fit_z != jnp.roll(fit_z, 1, axis=1))
    t = jnp.where(bnd, lane, jnp.int32(1 << 20))
    sfx = t
    for d in (1, 2, 4, 8):
        sh = jnp.where(lmod <= L - 1 - d, jnp.roll(sfx, -d, axis=1), jnp.int32(1 << 20))
        sfx = jnp.minimum(sfx, sh)
    nxt_in = jnp.where(lmod <= L - 2, jnp.roll(sfx, -1, axis=1), jnp.int32(1 << 20))
    inext = jnp.minimum(nxt_in, lane - lmod + L)
    icnt = (inext - lane).astype(jnp.float32)
    isum = fit_z * icnt
    return isum, icnt, inext


def _rank_sort_body(
    vrows_ref, vcols_ref, pos_ref, s_ref, isum_ref, icnt_ref, inext_ref
):
    lane = jax.lax.broadcasted_iota(jnp.int32, (16, 128), 0) * 128 + \
        jax.lax.broadcasted_iota(jnp.int32, (16, 128), 1)
    lmod = lane & (L - 1)
    lmod_f = lmod.astype(jnp.float32)
    jj = jax.lax.broadcasted_iota(jnp.int32, (BLK, N), 1)
    ii0 = jax.lax.broadcasted_iota(jnp.int32, (BLK, N), 0)
    for r in range(2):
        vrow = vrows_ref[r : r + 1, :]

        def blk(b, acc):
            vi = vcols_ref[pl.ds(b * BLK, BLK), r : r + 1]
            before = (vrow > vi) | ((vrow == vi) & (jj < ii0 + b * BLK))
            cnt = jnp.sum(jnp.where(before, 1.0, 0.0), axis=1, keepdims=True)
            pos_ref[pl.ds(b * BLK, BLK), r : r + 1] = cnt
            e = jnp.where(cnt.astype(jnp.int32) == jj, vi, 0.0)
            return acc + jnp.sum(e, axis=0, keepdims=True)

        s_row = jax.lax.fori_loop(
            0, NBLK, blk, jnp.zeros((1, N), jnp.float32)
        )
        s_ref[r : r + 1, :] = s_row
        s2 = s_row.reshape(16, 128)
        isum, icnt, inext = _chunk_items(s2, lane, lmod, lmod_f)
        isum_ref[r, :, :] = isum
        icnt_ref[r, :, :] = icnt
        inext_ref[r, :, :] = inext


def _pav_body(isum_ref, icnt_ref, inext_ref, bm_ref, bs_ref, bc_ref, nb_ref, means, counts):
    n = N

    for r in range(2):

        def cond(carry):
            return carry[4]

        def trip(carry):
            k, sp, top_s, top_c, _ = carry
            prev = jnp.maximum(sp - 1, 0)
            ps = means[r, prev]
            pc = counts[r, prev]
            can_merge = (sp > 0) & (top_s * pc > ps * top_c)
            can_push = k < n
            ms = top_s + ps
            mc = top_c + pc
            kc = jnp.minimum(k, n - 1)
            push_s = isum_ref[r, kc]
            push_c = icnt_ref[r, kc]
            push_k = inext_ref[r, kc]
            st_idx = jnp.where(can_merge, prev, jnp.where(can_push, sp, n))
            means[r, st_idx] = jnp.where(can_merge, ms, top_s)
            counts[r, st_idx] = jnp.where(can_merge, mc, top_c)
            new_top_s = jnp.where(can_merge, ms, jnp.where(can_push, push_s, top_s))
            new_top_c = jnp.where(can_merge, mc, jnp.where(can_push, push_c, top_c))
            new_sp = sp + jnp.where(can_merge, -1, jnp.where(can_push, 1, 0))
            new_k = jnp.where(can_merge, k, jnp.where(can_push, push_k, k))
            nprev = jnp.maximum(new_sp - 1, 0)
            nps = means[r, nprev]
            npc = counts[r, nprev]
            nactive = ((new_sp > 0) & (new_top_s * npc > nps * new_top_c)) | (
                new_k < n
            )
            return new_k, new_sp, new_top_s, new_top_c, nactive

        k0 = inext_ref[r, 0]
        init = (k0, jnp.int32(0), isum_ref[r, 0], icnt_ref[r, 0], k0 < n)
        k, sp, top_s, top_c, _ = jax.lax.while_loop(cond, trip, init)
        means[r, sp] = top_s
        counts[r, sp] = top_c
        nb_ref[0, r] = (sp + 1).astype(jnp.float32)

        def walk(b, start):
            c = counts[r, b]
            bm_ref[r, b] = means[r, b] / c
            bs_ref[r, b] = start
            bc_ref[r, b] = c
            return start + c

        jax.lax.fori_loop(0, sp + 1, walk, jnp.float32(0.0))


def _loss_body(s_ref, pos_ref, bm_ref, bs_ref, bc_ref, nb_ref, out_ref):
    fn = jnp.float32(N)
    nb0 = nb_ref[0, 0]
    nb1 = nb_ref[0, 1]
    kk_f = jax.lax.broadcasted_iota(jnp.int32, (BLK, N), 1).astype(jnp.float32)

    def dual_blk(b, accs):
        d0, d1 = accs
        bf = (
            jax.lax.broadcasted_iota(jnp.int32, (BLK, 1), 0) + b * BLK
        ).astype(jnp.float32)
        bs0 = bs_ref[pl.ds(b * BLK, BLK), 0:1]
        bc0 = bc_ref[pl.ds(b * BLK, BLK), 0:1]
        bm0 = bm_ref[pl.ds(b * BLK, BLK), 0:1]
        m0 = (bs0 <= kk_f) & (kk_f < bs0 + bc0) & (bf < nb0)
        d0 = d0 + jnp.sum(jnp.where(m0, bm0, 0.0), axis=0, keepdims=True)
        bs1 = bs_ref[pl.ds(b * BLK, BLK), 1:2]
        bc1 = bc_ref[pl.ds(b * BLK, BLK), 1:2]
        bm1 = bm_ref[pl.ds(b * BLK, BLK), 1:2]
        m1 = (bs1 <= kk_f) & (kk_f < bs1 + bc1) & (bf < nb1)
        d1 = d1 + jnp.sum(jnp.where(m1, bm1, 0.0), axis=0, keepdims=True)
        return d0, d1

    zero = jnp.zeros((1, N), jnp.float32)
    dual0, dual1 = jax.lax.fori_loop(0, NBLK, dual_blk, (zero, zero))
    solp = s_ref[0:1, :] - dual0
    solt = s_ref[1:2, :] - dual1
    mp = jnp.sum(solp) / fn
    mt = jnp.sum(solt) / fn
    vp = jnp.sum((solp - mp) ** 2)
    vt = jnp.sum((solt - mt) ** 2)
    cp = solp - mp
    ct = solt - mt

    def dot_blk(b, acc):
        kk = jax.lax.broadcasted_iota(jnp.int32, (BLK, N), 1)
        pp = pos_ref[pl.ds(b * BLK, BLK), 0:1].astype(jnp.int32)
        pt = pos_ref[pl.ds(b * BLK, BLK), 1:2].astype(jnp.int32)
        rp = jnp.sum(jnp.where(pp == kk, cp, 0.0), axis=1, keepdims=True)
        rt = jnp.sum(jnp.where(pt == kk, ct, 0.0), axis=1, keepdims=True)
        return acc + jnp.sum(rp * rt)

    dot = jax.lax.fori_loop(0, NBLK, dot_blk, jnp.float32(0.0))
    out_ref[0, 0] = -dot / (jnp.sqrt(vp) * jnp.sqrt(vt))


def kernel(pred, target):
    vrows = jnp.concatenate([pred, target], axis=0)
    vcols = vrows.T
    pos, s, isum, icnt, inext = pl.pallas_call(
        _rank_sort_body,
        out_shape=(
            jax.ShapeDtypeStruct((N, 2), jnp.float32),
            jax.ShapeDtypeStruct((2, N), jnp.float32),
            jax.ShapeDtypeStruct((2, 16, 128), jnp.float32),
            jax.ShapeDtypeStruct((2, 16, 128), jnp.float32),
            jax.ShapeDtypeStruct((2, 16, 128), jnp.int32),
        ),
    )(vrows, vcols)
    isum = isum.reshape(2, N)
    icnt = icnt.reshape(2, N)
    inext = inext.reshape(2, N)
    bm, bs, bc, nb = pl.pallas_call(
        _pav_body,
        in_specs=[pl.BlockSpec(memory_space=pltpu.SMEM)] * 3,
        out_specs=tuple([pl.BlockSpec(memory_space=pltpu.SMEM)] * 4),
        out_shape=(
            jax.ShapeDtypeStruct((2, N), jnp.float32),
            jax.ShapeDtypeStruct((2, N), jnp.float32),
            jax.ShapeDtypeStruct((2, N), jnp.float32),
            jax.ShapeDtypeStruct((1, 2), jnp.float32),
        ),
        scratch_shapes=[
            pltpu.SMEM((2, N + 1), jnp.float32),
            pltpu.SMEM((2, N + 1), jnp.float32),
        ],
    )(isum, icnt, inext)
    loss = pl.pallas_call(
        _loss_body,
        in_specs=[pl.BlockSpec(memory_space=pltpu.VMEM)] * 5
        + [pl.BlockSpec(memory_space=pltpu.SMEM)],
        out_specs=pl.BlockSpec(memory_space=pltpu.SMEM),
        out_shape=jax.ShapeDtypeStruct((1, 1), jnp.float32),
    )(s, pos, bm.T, bs.T, bc.T, nb)
    return loss[0, 0]


# speedup vs baseline: 39.2791x; 1.4357x over previous
import jax
import jax.numpy as jnp
from jax.experimental import pallas as pl
from jax.experimental.pallas import tpu as pltpu

N = 2048
BLK = 256
NBLK = N // BLK
L = 16
BIG = 1e30


def _chunk_items(s2, lane, lmod, lmod_f):
    lane_f = lane.astype(jnp.float32)
    y = (jnp.float32(N) - lane_f) - s2
    cs = y
    for d in (1, 2, 4, 8):
        cs = jnp.where(lmod >= d, cs + jnp.roll(cs, d, axis=1), cs)
    cs_excl = cs - y
    fit_y = jnp.full(s2.shape, -BIG, jnp.float32)
    for i_off in range(L):
        f = jnp.where(lmod == i_off, cs_excl, 0.0)
        for d in (1, 2, 4, 8):
            f = jnp.where(lmod >= d, f + jnp.roll(f, d, axis=1), f)
        len_f = lmod_f - jnp.float32(i_off) + 1.0
        mj = (cs - f) / len_f
        mj = jnp.where(lmod >= i_off, mj, BIG)
        for d in (1, 2, 4, 8):
            sh = jnp.where(lmod <= L - 1 - d, jnp.roll(mj, -d, axis=1), BIG)
            mj = jnp.minimum(mj, sh)
        fit_y = jnp.maximum(fit_y, jnp.where(lmod >= i_off, mj, -BIG))
    fit_z = -fit_y
    bnd = (lmod == 0) | (fit_z != jnp.roll(fit_z, 1, axis=1))
    t = jnp.where(bnd, lane, jnp.int32(1 << 20))
    sfx = t
    for d in (1, 2, 4, 8):
        sh = jnp.where(lmod <= L - 1 - d, jnp.roll(sfx, -d, axis=1), jnp.int32(1 << 20))
        sfx = jnp.minimum(sfx, sh)
    nxt_in = jnp.where(lmod <= L - 2, jnp.roll(sfx, -1, axis=1), jnp.int32(1 << 20))
    inext = jnp.minimum(nxt_in, lane - lmod + L)
    icnt = (inext - lane).astype(jnp.float32)
    isum = fit_z * icnt
    return isum, icnt, inext


def _rank_sort_body(
    vrows_ref, vcols_ref, pos_ref, s_ref, isum_ref, icnt_ref, inext_ref
):
    lane = jax.lax.broadcasted_iota(jnp.int32, (16, 128), 0) * 128 + \
        jax.lax.broadcasted_iota(jnp.int32, (16, 128), 1)
    lmod = lane & (L - 1)
    lmod_f = lmod.astype(jnp.float32)
    jj = jax.lax.broadcasted_iota(jnp.int32, (BLK, N), 1)
    ii0 = jax.lax.broadcasted_iota(jnp.int32, (BLK, N), 0)
    for r in range(2):
        vrow = vrows_ref[r : r + 1, :]

        def blk(b, acc):
            vi = vcols_ref[pl.ds(b * BLK, BLK), r : r + 1]
            before = (vrow > vi) | ((vrow == vi) & (jj < ii0 + b * BLK))
            cnt = jnp.sum(jnp.where(before, 1.0, 0.0), axis=1, keepdims=True)
            pos_ref[pl.ds(b * BLK, BLK), r : r + 1] = cnt
            e = jnp.where(cnt.astype(jnp.int32) == jj, vi, 0.0)
            return acc + jnp.sum(e, axis=0, keepdims=True)

        s_row = jax.lax.fori_loop(
            0, NBLK, blk, jnp.zeros((1, N), jnp.float32)
        )
        s_ref[r : r + 1, :] = s_row
        s2 = s_row.reshape(16, 128)
        isum, icnt, inext = _chunk_items(s2, lane, lmod, lmod_f)
        isum_ref[r, :, :] = isum
        icnt_ref[r, :, :] = icnt
        inext_ref[r, :, :] = inext


def _pav_loss_body(s_ref, pos_ref, isum_ref, icnt_ref, inext_ref, out_ref, means, counts):
    n = N
    fn = jnp.float32(N)
    lane_f = (
        jax.lax.broadcasted_iota(jnp.int32, (1, N), 1).astype(jnp.float32)
    )
    sols = []
    for r in range(2):

        def cond(carry):
            return carry[4]

        def trip(carry):
            k, sp, top_s, top_c, _ = carry
            prev = jnp.maximum(sp - 1, 0)
            ps = means[r, prev]
            pc = counts[r, prev]
            can_merge = (sp > 0) & (top_s * pc > ps * top_c)
            can_push = k < n
            ms = top_s + ps
            mc = top_c + pc
            kc = jnp.minimum(k, n - 1)
            push_s = isum_ref[r, kc]
            push_c = icnt_ref[r, kc]
            push_k = inext_ref[r, kc]
            st_idx = jnp.where(can_merge, prev, jnp.where(can_push, sp, n))
            means[r, st_idx] = jnp.where(can_merge, ms, top_s)
            counts[r, st_idx] = jnp.where(can_merge, mc, top_c)
            new_top_s = jnp.where(can_merge, ms, jnp.where(can_push, push_s, top_s))
            new_top_c = jnp.where(can_merge, mc, jnp.where(can_push, push_c, top_c))
            new_sp = sp + jnp.where(can_merge, -1, jnp.where(can_push, 1, 0))
            new_k = jnp.where(can_merge, k, jnp.where(can_push, push_k, k))
            nprev = jnp.maximum(new_sp - 1, 0)
            nps = means[r, nprev]
            npc = counts[r, nprev]
            nactive = ((new_sp > 0) & (new_top_s * npc > nps * new_top_c)) | (
                new_k < n
            )
            return new_k, new_sp, new_top_s, new_top_c, nactive

        k0 = inext_ref[r, 0]
        init = (k0, jnp.int32(0), isum_ref[r, 0], icnt_ref[r, 0], k0 < n)
        k, sp, top_s, top_c, _ = jax.lax.while_loop(cond, trip, init)
        means[r, sp] = top_s
        counts[r, sp] = top_c

        def fill(b, carry):
            start, dual = carry
            c = counts[r, b]
            m = means[r, b] / c
            dual = dual + jnp.where(
                (lane_f >= start) & (lane_f < start + c), m, 0.0
            )
            return start + c, dual

        _, dual = jax.lax.fori_loop(
            0, sp + 1, fill, (jnp.float32(0.0), jnp.zeros((1, N), jnp.float32))
        )
        sols.append(s_ref[r : r + 1, :] - dual)

    solp, solt = sols
    mp = jnp.sum(solp) / fn
    mt = jnp.sum(solt) / fn
    vp = jnp.sum((solp - mp) ** 2)
    vt = jnp.sum((solt - mt) ** 2)
    cp = solp - mp
    ct = solt - mt

    def dot_blk(b, acc):
        kk = jax.lax.broadcasted_iota(jnp.int32, (BLK, N), 1)
        pp = pos_ref[pl.ds(b * BLK, BLK), 0:1].astype(jnp.int32)
        pt = pos_ref[pl.ds(b * BLK, BLK), 1:2].astype(jnp.int32)
        rp = jnp.sum(jnp.where(pp == kk, cp, 0.0), axis=1, keepdims=True)
        rt = jnp.sum(jnp.where(pt == kk, ct, 0.0), axis=1, keepdims=True)
        return acc + jnp.sum(rp * rt)

    dot = jax.lax.fori_loop(0, NBLK, dot_blk, jnp.float32(0.0))
    out_ref[0, 0] = -dot / (jnp.sqrt(vp) * jnp.sqrt(vt))


def kernel(pred, target):
    vrows = jnp.concatenate([pred, target], axis=0)
    vcols = vrows.T
    pos, s, isum, icnt, inext = pl.pallas_call(
        _rank_sort_body,
        out_shape=(
            jax.ShapeDtypeStruct((N, 2), jnp.float32),
            jax.ShapeDtypeStruct((2, N), jnp.float32),
            jax.ShapeDtypeStruct((2, 16, 128), jnp.float32),
            jax.ShapeDtypeStruct((2, 16, 128), jnp.float32),
            jax.ShapeDtypeStruct((2, 16, 128), jnp.int32),
        ),
    )(vrows, vcols)
    isum = isum.reshape(2, N)
    icnt = icnt.reshape(2, N)
    inext = inext.reshape(2, N)
    loss = pl.pallas_call(
        _pav_loss_body,
        in_specs=[pl.BlockSpec(memory_space=pltpu.VMEM)] * 2
        + [pl.BlockSpec(memory_space=pltpu.SMEM)] * 3,
        out_specs=pl.BlockSpec(memory_space=pltpu.SMEM),
        out_shape=jax.ShapeDtypeStruct((1, 1), jnp.float32),
        scratch_shapes=[
            pltpu.SMEM((2, N + 1), jnp.float32),
            pltpu.SMEM((2, N + 1), jnp.float32),
        ],
    )(s, pos, isum, icnt, inext)
    return loss[0, 0]


# speedup vs baseline: 39.8226x; 1.0138x over previous
import jax
import jax.numpy as jnp
from jax.experimental import pallas as pl
from jax.experimental.pallas import tpu as pltpu

N = 2048
BLK = 512
NBLK = N // BLK
L = 16
BIG = 1e30


def _chunk_items(s2, lane, lmod, lmod_f):
    lane_f = lane.astype(jnp.float32)
    y = (jnp.float32(N) - lane_f) - s2
    cs = y
    for d in (1, 2, 4, 8):
        cs = jnp.where(lmod >= d, cs + jnp.roll(cs, d, axis=1), cs)
    cs_excl = cs - y
    fit_y = jnp.full(s2.shape, -BIG, jnp.float32)
    for i_off in range(L):
        f = jnp.where(lmod == i_off, cs_excl, 0.0)
        for d in (1, 2, 4, 8):
            f = jnp.where(lmod >= d, f + jnp.roll(f, d, axis=1), f)
        len_f = lmod_f - jnp.float32(i_off) + 1.0
        mj = (cs - f) / len_f
        mj = jnp.where(lmod >= i_off, mj, BIG)
        for d in (1, 2, 4, 8):
            sh = jnp.where(lmod <= L - 1 - d, jnp.roll(mj, -d, axis=1), BIG)
            mj = jnp.minimum(mj, sh)
        fit_y = jnp.maximum(fit_y, jnp.where(lmod >= i_off, mj, -BIG))
    fit_z = -fit_y
    bnd = (lmod == 0) | (fit_z != jnp.roll(fit_z, 1, axis=1))
    t = jnp.where(bnd, lane, jnp.int32(1 << 20))
    sfx = t
    for d in (1, 2, 4, 8):
        sh = jnp.where(lmod <= L - 1 - d, jnp.roll(sfx, -d, axis=1), jnp.int32(1 << 20))
        sfx = jnp.minimum(sfx, sh)
    nxt_in = jnp.where(lmod <= L - 2, jnp.roll(sfx, -1, axis=1), jnp.int32(1 << 20))
    inext = jnp.minimum(nxt_in, lane - lmod + L)
    icnt = (inext - lane).astype(jnp.float32)
    isum = fit_z * icnt
    return isum, icnt, inext


def _rank_sort_body(
    vrows_ref, vcols_ref, pos_ref, s_ref, isum_ref, icnt_ref, inext_ref
):
    lane = jax.lax.broadcasted_iota(jnp.int32, (16, 128), 0) * 128 + \
        jax.lax.broadcasted_iota(jnp.int32, (16, 128), 1)
    lmod = lane & (L - 1)
    lmod_f = lmod.astype(jnp.float32)
    jj = jax.lax.broadcasted_iota(jnp.int32, (BLK, N), 1)
    ii0 = jax.lax.broadcasted_iota(jnp.int32, (BLK, N), 0)
    for r in range(2):
        vrow = vrows_ref[r : r + 1, :]

        def blk(b, acc):
            vi = vcols_ref[pl.ds(b * BLK, BLK), r : r + 1]
            before = (vrow > vi) | ((vrow == vi) & (jj < ii0 + b * BLK))
            cnt = jnp.sum(jnp.where(before, 1.0, 0.0), axis=1, keepdims=True)
            pos_ref[pl.ds(b * BLK, BLK), r : r + 1] = cnt
            e = jnp.where(cnt.astype(jnp.int32) == jj, vi, 0.0)
            return acc + jnp.sum(e, axis=0, keepdims=True)

        s_row = jax.lax.fori_loop(
            0, NBLK, blk, jnp.zeros((1, N), jnp.float32)
        )
        s_ref[r : r + 1, :] = s_row
        s2 = s_row.reshape(16, 128)
        isum, icnt, inext = _chunk_items(s2, lane, lmod, lmod_f)
        isum_ref[r, :, :] = isum
        icnt_ref[r, :, :] = icnt
        inext_ref[r, :, :] = inext


def _pav_loss_body(s_ref, pos_ref, isum_ref, icnt_ref, inext_ref, out_ref, means, counts):
    n = N
    fn = jnp.float32(N)
    lane_f = (
        jax.lax.broadcasted_iota(jnp.int32, (1, N), 1).astype(jnp.float32)
    )
    sols = []
    for r in range(2):

        def cond(carry):
            return carry[4]

        def trip(carry):
            k, sp, top_s, top_c, _ = carry
            prev = jnp.maximum(sp - 1, 0)
            ps = means[r, prev]
            pc = counts[r, prev]
            can_merge = (sp > 0) & (top_s * pc > ps * top_c)
            can_push = k < n
            ms = top_s + ps
            mc = top_c + pc
            kc = jnp.minimum(k, n - 1)
            push_s = isum_ref[r, kc]
            push_c = icnt_ref[r, kc]
            push_k = inext_ref[r, kc]
            st_idx = jnp.where(can_merge, prev, jnp.where(can_push, sp, n))
            means[r, st_idx] = jnp.where(can_merge, ms, top_s)
            counts[r, st_idx] = jnp.where(can_merge, mc, top_c)
            new_top_s = jnp.where(can_merge, ms, jnp.where(can_push, push_s, top_s))
            new_top_c = jnp.where(can_merge, mc, jnp.where(can_push, push_c, top_c))
            new_sp = sp + jnp.where(can_merge, -1, jnp.where(can_push, 1, 0))
            new_k = jnp.where(can_merge, k, jnp.where(can_push, push_k, k))
            nprev = jnp.maximum(new_sp - 1, 0)
            nps = means[r, nprev]
            npc = counts[r, nprev]
            nactive = ((new_sp > 0) & (new_top_s * npc > nps * new_top_c)) | (
                new_k < n
            )
            return new_k, new_sp, new_top_s, new_top_c, nactive

        k0 = inext_ref[r, 0]
        init = (k0, jnp.int32(0), isum_ref[r, 0], icnt_ref[r, 0], k0 < n)
        k, sp, top_s, top_c, _ = jax.lax.while_loop(cond, trip, init)
        means[r, sp] = top_s
        counts[r, sp] = top_c

        def fill(b, carry):
            start, dual = carry
            c = counts[r, b]
            m = means[r, b] / c
            dual = dual + jnp.where(
                (lane_f >= start) & (lane_f < start + c), m, 0.0
            )
            return start + c, dual

        _, dual = jax.lax.fori_loop(
            0, sp + 1, fill, (jnp.float32(0.0), jnp.zeros((1, N), jnp.float32))
        )
        sols.append(s_ref[r : r + 1, :] - dual)

    solp, solt = sols
    mp = jnp.sum(solp) / fn
    mt = jnp.sum(solt) / fn
    vp = jnp.sum((solp - mp) ** 2)
    vt = jnp.sum((solt - mt) ** 2)
    cp = solp - mp
    ct = solt - mt

    def dot_blk(b, acc):
        kk = jax.lax.broadcasted_iota(jnp.int32, (BLK, N), 1)
        pp = pos_ref[pl.ds(b * BLK, BLK), 0:1].astype(jnp.int32)
        pt = pos_ref[pl.ds(b * BLK, BLK), 1:2].astype(jnp.int32)
        rp = jnp.sum(jnp.where(pp == kk, cp, 0.0), axis=1, keepdims=True)
        rt = jnp.sum(jnp.where(pt == kk, ct, 0.0), axis=1, keepdims=True)
        return acc + jnp.sum(rp * rt)

    dot = jax.lax.fori_loop(0, NBLK, dot_blk, jnp.float32(0.0))
    out_ref[0, 0] = -dot / (jnp.sqrt(vp) * jnp.sqrt(vt))


def kernel(pred, target):
    vrows = jnp.concatenate([pred, target], axis=0)
    vcols = vrows.T
    pos, s, isum, icnt, inext = pl.pallas_call(
        _rank_sort_body,
        out_shape=(
            jax.ShapeDtypeStruct((N, 2), jnp.float32),
            jax.ShapeDtypeStruct((2, N), jnp.float32),
            jax.ShapeDtypeStruct((2, 16, 128), jnp.float32),
            jax.ShapeDtypeStruct((2, 16, 128), jnp.float32),
            jax.ShapeDtypeStruct((2, 16, 128), jnp.int32),
        ),
    )(vrows, vcols)
    isum = isum.reshape(2, N)
    icnt = icnt.reshape(2, N)
    inext = inext.reshape(2, N)
    loss = pl.pallas_call(
        _pav_loss_body,
        in_specs=[pl.BlockSpec(memory_space=pltpu.VMEM)] * 2
        + [pl.BlockSpec(memory_space=pltpu.SMEM)] * 3,
        out_specs=pl.BlockSpec(memory_space=pltpu.SMEM),
        out_shape=jax.ShapeDtypeStruct((1, 1), jnp.float32),
        scratch_shapes=[
            pltpu.SMEM((2, N + 1), jnp.float32),
            pltpu.SMEM((2, N + 1), jnp.float32),
        ],
    )(s, pos, isum, icnt, inext)
    return loss[0, 0]
